# baseline (speedup 1.0000x reference)
import functools

import jax
import jax.numpy as jnp
from jax import lax
from jax.experimental import pallas as pl
from jax.experimental.pallas import tpu as pltpu
from jax.experimental.pallas import tpu_sc as plsc

N = 10000
NH = 128
E = 160000
NT = 10112
ZR = N
CH = 128
TPC = 80
EPAD = 16 * TPC * CH
NACC = 10112
RPT = NACC // 16
IB = 40

_f32 = jnp.float32


def _make_edge_pass(name):
    mesh = plsc.VectorSubcoreMesh(core_axis_name="c", subcore_axis_name="s")

    @functools.partial(
        pl.kernel,
        out_type=jax.ShapeDtypeStruct((2, N, 128), _f32),
        mesh=mesh,
        scratch_types=[
            pltpu.VMEM((IB, CH), jnp.int32),
            pltpu.VMEM((IB, CH), jnp.int32),
            pltpu.VMEM((CH, 128), _f32),
            pltpu.VMEM((CH, 128), _f32),
            pltpu.VMEM_SHARED((NACC, 128), _f32),
            pltpu.SemaphoreType.DMA,
            pltpu.SemaphoreType.DMA,
        ],
        name=name,
    )
    def edge_pass(tab0, tab1, zeros_hbm, rowidx, colidx, out,
                  row_v, col_v, buf0, buf1, acc, sem0, sem1):
        c = lax.axis_index("c")
        s = lax.axis_index("s")
        base = s * RPT
        pltpu.sync_copy(zeros_hbm, acc.at[pl.ds(base, RPT)])
        plsc.subcore_barrier()

        def run(tab):
            for b in range(TPC // IB):
                pltpu.sync_copy(rowidx.at[s, pl.ds(b * IB, IB)], row_v)
                pltpu.sync_copy(colidx.at[s, pl.ds(b * IB, IB)], col_v)
                pltpu.async_copy(tab.at[col_v.at[0]], buf0, sem0)

                def body(k, carry):
                    j0 = 2 * k
                    h1 = pltpu.async_copy(tab.at[col_v.at[j0 + 1]], buf1, sem1)
                    pltpu.make_async_copy(tab.at[pl.ds(0, CH)], buf0, sem0).wait()
                    pltpu.sync_copy(buf0, acc.at[row_v.at[j0]], add=True)

                    @pl.when(k < IB // 2 - 1)
                    def _():
                        pltpu.async_copy(tab.at[col_v.at[j0 + 2]], buf0, sem0)

                    h1.wait()
                    pltpu.sync_copy(buf1, acc.at[row_v.at[j0 + 1]], add=True)
                    return carry

                lax.fori_loop(0, IB // 2, body, 0)

        @pl.when(c == 0)
        def _():
            run(tab0)

        @pl.when(c == 1)
        def _():
            run(tab1)

        plsc.subcore_barrier()

        @pl.when(s < 15)
        def _():
            pltpu.sync_copy(acc.at[pl.ds(base, RPT)], out.at[c, pl.ds(base, RPT)])

        @pl.when(s == 15)
        def _():
            pltpu.sync_copy(acc.at[pl.ds(15 * RPT, N - 15 * RPT)],
                            out.at[c, pl.ds(15 * RPT, N - 15 * RPT)])

    return edge_pass


_edge_pass_att = _make_edge_pass("att_edge_pass")


def _make_split_pass(name):
    mesh = plsc.VectorSubcoreMesh(core_axis_name="c", subcore_axis_name="s")

    @functools.partial(
        pl.kernel,
        out_type=jax.ShapeDtypeStruct((2, N, 128), _f32),
        mesh=mesh,
        scratch_types=[
            pltpu.VMEM((IB, CH), jnp.int32),
            pltpu.VMEM((IB, CH), jnp.int32),
            pltpu.VMEM((CH, 128), _f32),
            pltpu.VMEM((CH, 128), _f32),
            pltpu.VMEM_SHARED((NACC, 128), _f32),
            pltpu.SemaphoreType.DMA,
            pltpu.SemaphoreType.DMA,
        ],
        name=name,
    )
    def scalar_pass(tab, zeros_hbm, rowidx, colidx, out,
                    row_v, col_v, buf0, buf1, acc, sem0, sem1):
        c = lax.axis_index("c")
        s = lax.axis_index("s")
        base = s * RPT
        pltpu.sync_copy(zeros_hbm, acc.at[pl.ds(base, RPT)])
        plsc.subcore_barrier()

        pltpu.sync_copy(rowidx.at[c, s], row_v)
        pltpu.sync_copy(colidx.at[c, s], col_v)
        pltpu.async_copy(tab.at[col_v.at[0]], buf0, sem0)

        def body(k, carry):
            j0 = 2 * k
            h1 = pltpu.async_copy(tab.at[col_v.at[j0 + 1]], buf1, sem1)
            pltpu.make_async_copy(tab.at[pl.ds(0, CH)], buf0, sem0).wait()
            pltpu.sync_copy(buf0, acc.at[row_v.at[j0]], add=True)

            @pl.when(k < IB // 2 - 1)
            def _():
                pltpu.async_copy(tab.at[col_v.at[j0 + 2]], buf0, sem0)

            h1.wait()
            pltpu.sync_copy(buf1, acc.at[row_v.at[j0 + 1]], add=True)
            return carry

        lax.fori_loop(0, IB // 2, body, 0)

        plsc.subcore_barrier()

        @pl.when(s < 15)
        def _():
            pltpu.sync_copy(acc.at[pl.ds(base, RPT)], out.at[c, pl.ds(base, RPT)])

        @pl.when(s == 15)
        def _():
            pltpu.sync_copy(acc.at[pl.ds(15 * RPT, N - 15 * RPT)],
                            out.at[c, pl.ds(15 * RPT, N - 15 * RPT)])

    return scalar_pass


_gcn_pass = _make_split_pass("gcn_edge_pass")
_scalar_pass = _make_split_pass("scalar_edge_pass")


def _k3_body(aggx_ref, gcw_ref, gctw_ref, gcb_ref, gctb_ref, a_ref,
             tabu_ref, tabt_ref, tabs_ref, repo_ref, g_ref, misc_ref):
    aggx = aggx_ref[0] + aggx_ref[1]
    rep_o = jnp.maximum(
        jnp.dot(aggx, gcw_ref[...], preferred_element_type=_f32)
        + gcb_ref[...], 0.0)
    rep_t = jnp.maximum(
        jnp.dot(aggx, gctw_ref[...], preferred_element_type=_f32)
        + gctb_ref[...], 0.0)
    a = a_ref[...]
    s1 = (jnp.dot(rep_o, a[0:128], preferred_element_type=_f32)
          + jnp.dot(rep_t, a[128:256], preferred_element_type=_f32))
    s2 = (jnp.dot(rep_o, a[256:384], preferred_element_type=_f32)
          + jnp.dot(rep_t, a[384:512], preferred_element_type=_f32))
    c = jnp.max(s2)
    e2 = jnp.exp(s2 - c)
    u = e2 * rep_t
    cid = lax.broadcasted_iota(jnp.int32, (N, 128), 1)
    sc = jnp.where(cid == 0, e2, jnp.where(cid == 1, 1.0, 0.0))
    zpad = jnp.zeros((NT - N, 128), _f32)
    tabu_ref[0:N, :] = u
    tabu_ref[N:NT, :] = zpad
    tabt_ref[0:N, :] = rep_t
    tabt_ref[N:NT, :] = zpad
    tabs_ref[0:N, :] = sc
    tabs_ref[N:NT, :] = zpad
    repo_ref[...] = rep_o
    g_ref[...] = s1 + c
    t_sum = jnp.sum(rep_t, axis=0, keepdims=True)
    misc_ref[...] = jnp.broadcast_to(t_sum, (8, 128))


def _k3(aggx, gc_W, gct_W, gc_b, gct_b, a):
    return pl.pallas_call(
        _k3_body,
        out_shape=[
            jax.ShapeDtypeStruct((NT, 128), _f32),
            jax.ShapeDtypeStruct((NT, 128), _f32),
            jax.ShapeDtypeStruct((NT, 128), _f32),
            jax.ShapeDtypeStruct((N, 128), _f32),
            jax.ShapeDtypeStruct((N, 1), _f32),
            jax.ShapeDtypeStruct((8, 128), _f32),
        ],
    )(aggx, gc_W, gct_W, gc_b.reshape(1, 128), gct_b.reshape(1, 128), a)


def _k5_body(att_ref, att2_ref, rept_ref, repo_ref, g_ref, misc_ref, t_ref,
             ppw_ref, ppb_ref, pp2w_ref, pp2b_ref,
             o00w_ref, o00b_ref, o10w_ref, o10b_ref,
             o01w_ref, o01b_ref, o11w_ref, o11b_ref,
             y_ref, out2_ref, treat_ref):
    v = att_ref[0]
    wm = att_ref[1]
    z = att2_ref[0, :, 0:1] + att2_ref[1, :, 0:1]
    deg = att2_ref[0, :, 1:2] + att2_ref[1, :, 1:2]
    g = g_ref[...]
    m = jnp.maximum(g, 0.0)
    amp = jnp.exp(0.0 - m)
    bmp = jnp.exp(g - m)
    t_sum = misc_ref[0:1, :]
    denom = (float(N) - deg) * amp + bmp * z
    out2 = (amp * (t_sum - wm) + bmp * v) / denom + repo_ref[...]
    out2_ref[...] = out2
    rep_t = rept_ref[...]
    tm = jnp.dot(rep_t, ppw_ref[...], preferred_element_type=_f32) + ppb_ref[...]
    logits = jnp.dot(tm, pp2w_ref[...], preferred_element_type=_f32) + pp2b_ref[...]
    treat_ref[...] = jax.nn.sigmoid(logits)
    y00 = jnp.maximum(jnp.dot(out2, o00w_ref[...], preferred_element_type=_f32)
                      + o00b_ref[...], 0.0)
    y0 = jnp.dot(y00, o01w_ref[...], preferred_element_type=_f32) + o01b_ref[...]
    y10 = jnp.maximum(jnp.dot(out2, o10w_ref[...], preferred_element_type=_f32)
                      + o10b_ref[...], 0.0)
    y1 = jnp.dot(y10, o11w_ref[...], preferred_element_type=_f32) + o11b_ref[...]
    y_ref[...] = jnp.where(t_ref[...] > 0, y1, y0)


def _k5(att, att2, rep_t, rep_o, g, misc, t2, pp_W, pp_b, pp2_W, pp2_b,
        o00_W, o00_b, o10_W, o10_b, o01_W, o01_b, o11_W, o11_b):
    nb = 10
    br = N // nb
    row_specs = lambda wdt: pl.BlockSpec((br, wdt), lambda i: (i, 0))
    pair_spec = pl.BlockSpec((2, br, 128), lambda i: (0, i, 0))
    full = lambda shape: pl.BlockSpec(shape, lambda i: tuple(0 for _ in shape))
    return pl.pallas_call(
        _k5_body,
        grid=(nb,),
        in_specs=[
            pair_spec, pair_spec,
            row_specs(128), row_specs(128), row_specs(1),
            full((8, 128)), row_specs(1),
            full((128, 128)), full((1, 128)), full((128, 2)), full((1, 2)),
            full((128, 128)), full((1, 128)), full((128, 128)), full((1, 128)),
            full((128, 1)), full((1, 1)), full((128, 1)), full((1, 1)),
        ],
        out_specs=[row_specs(1), row_specs(128), row_specs(2)],
        out_shape=[
            jax.ShapeDtypeStruct((N, 1), _f32),
            jax.ShapeDtypeStruct((N, 128), _f32),
            jax.ShapeDtypeStruct((N, 2), _f32),
        ],
    )(att, att2, rep_t, rep_o, g, misc, t2,
      pp_W, pp_b.reshape(1, 128), pp2_W, pp2_b.reshape(1, 2),
      o00_W, o00_b.reshape(1, 128), o10_W, o10_b.reshape(1, 128),
      o01_W, o01_b.reshape(1, 1), o11_W, o11_b.reshape(1, 1))


def kernel(x, edge_index, t, gc_W, gc_b, gct_W, gct_b, a, pp_W, pp_b, pp2_W, pp2_b, o00_W, o00_b, o10_W, o10_b, o01_W, o01_b, o11_W, o11_b):
    row = edge_index[0]
    col = edge_index[1]
    npad = EPAD - E

    pad_idx = N + (jnp.arange(npad, dtype=jnp.int32) % (NT - N))
    trash_e = N + (jnp.arange(E, dtype=jnp.int32) % (NT - N))

    row_raw = jnp.concatenate([row, pad_idx])
    col_raw = jnp.concatenate([col, pad_idx])
    rowidx_gcn = row_raw.reshape(2, 16, IB, CH)
    col_gcn = col_raw.reshape(2, 16, IB, CH)
    xpad = jnp.concatenate([x, jnp.zeros((NT - N, 128), _f32)])

    key = (row * N + col
           + lax.convert_element_type(xpad[N, 0], jnp.int32)
           + jnp.minimum(row_raw[EPAD - 1], 0)
           + jnp.minimum(col_raw[EPAD - 1], 0))
    ks = jnp.sort(key)
    rs = ks // N
    cs = ks - rs * N
    dup = jnp.concatenate([jnp.zeros((1,), bool), ks[1:] == ks[:-1]])
    cse = jnp.where(dup, trash_e, cs)

    rs_p = jnp.concatenate([rs, pad_idx])
    cse_p = jnp.concatenate([cse, pad_idx])
    rowidx = rs_p.reshape(16, TPC, CH)
    col_att = cse_p.reshape(16, TPC, CH)
    rowidx_h = rs_p.reshape(2, 16, IB, CH)
    col_att_h = cse_p.reshape(2, 16, IB, CH)

    zeros128 = jnp.zeros((RPT, 128), _f32)

    aggx = _gcn_pass(xpad, zeros128, rowidx_gcn, col_gcn)
    tab_u, tab_t, tab_s, rep_o, g, misc = _k3(aggx, gc_W, gct_W, gc_b, gct_b, a)
    att = _edge_pass_att(tab_u, tab_t, zeros128, rowidx, col_att)
    att2 = _scalar_pass(tab_s, zeros128, rowidx_h, col_att_h)

    rep_t = tab_t[0:N, :]
    y2, out2, treat = _k5(att, att2, rep_t, rep_o, g, misc, t.reshape(N, 1),
                          pp_W, pp_b, pp2_W, pp2_b,
                          o00_W, o00_b, o10_W, o10_b,
                          o01_W, o01_b, o11_W, o11_b)
    return (y2.reshape(-1), out2, treat)


# speedup vs baseline: 1.0013x; 1.0013x over previous
import functools

import jax
import jax.numpy as jnp
from jax import lax
from jax.experimental import pallas as pl
from jax.experimental.pallas import tpu as pltpu
from jax.experimental.pallas import tpu_sc as plsc

N = 10000
NH = 128
E = 160000
NT = 10112
ZR = N
CH = 128
TPC = 80
EPAD = 16 * TPC * CH
NACC = 10112
RPT = NACC // 16
IB = 40

_f32 = jnp.float32


def _make_edge_pass(name):
    mesh = plsc.VectorSubcoreMesh(core_axis_name="c", subcore_axis_name="s")

    @functools.partial(
        pl.kernel,
        out_type=jax.ShapeDtypeStruct((2, N, 128), _f32),
        mesh=mesh,
        scratch_types=[
            pltpu.VMEM((IB, CH), jnp.int32),
            pltpu.VMEM((IB, CH), jnp.int32),
            pltpu.VMEM((CH, 128), _f32),
            pltpu.VMEM((CH, 128), _f32),
            pltpu.VMEM_SHARED((NACC, 128), _f32),
            pltpu.SemaphoreType.DMA,
            pltpu.SemaphoreType.DMA,
        ],
        name=name,
    )
    def edge_pass(tab0, tab1, zeros_hbm, rowidx, colidx, out,
                  row_v, col_v, buf0, buf1, acc, sem0, sem1):
        c = lax.axis_index("c")
        s = lax.axis_index("s")
        base = s * RPT
        pltpu.sync_copy(zeros_hbm, acc.at[pl.ds(base, RPT)])
        plsc.subcore_barrier()

        def run(tab):
            for b in range(TPC // IB):
                pltpu.sync_copy(rowidx.at[s, pl.ds(b * IB, IB)], row_v)
                pltpu.sync_copy(colidx.at[s, pl.ds(b * IB, IB)], col_v)
                pltpu.async_copy(tab.at[col_v.at[0]], buf0, sem0)

                def body(k, carry):
                    j0 = 2 * k
                    h1 = pltpu.async_copy(tab.at[col_v.at[j0 + 1]], buf1, sem1)
                    pltpu.make_async_copy(tab.at[pl.ds(0, CH)], buf0, sem0).wait()
                    pltpu.sync_copy(buf0, acc.at[row_v.at[j0]], add=True)

                    @pl.when(k < IB // 2 - 1)
                    def _():
                        pltpu.async_copy(tab.at[col_v.at[j0 + 2]], buf0, sem0)

                    h1.wait()
                    pltpu.sync_copy(buf1, acc.at[row_v.at[j0 + 1]], add=True)
                    return carry

                lax.fori_loop(0, IB // 2, body, 0)

        @pl.when(c == 0)
        def _():
            run(tab0)

        @pl.when(c == 1)
        def _():
            run(tab1)

        plsc.subcore_barrier()

        @pl.when(s < 15)
        def _():
            pltpu.sync_copy(acc.at[pl.ds(base, RPT)], out.at[c, pl.ds(base, RPT)])

        @pl.when(s == 15)
        def _():
            pltpu.sync_copy(acc.at[pl.ds(15 * RPT, N - 15 * RPT)],
                            out.at[c, pl.ds(15 * RPT, N - 15 * RPT)])

    return edge_pass


_edge_pass_att = _make_edge_pass("att_edge_pass")


def _make_split_pass(name):
    mesh = plsc.VectorSubcoreMesh(core_axis_name="c", subcore_axis_name="s")

    @functools.partial(
        pl.kernel,
        out_type=jax.ShapeDtypeStruct((2, N, 128), _f32),
        mesh=mesh,
        scratch_types=[
            pltpu.VMEM((IB, CH), jnp.int32),
            pltpu.VMEM((IB, CH), jnp.int32),
            pltpu.VMEM((CH, 128), _f32),
            pltpu.VMEM((CH, 128), _f32),
            pltpu.VMEM_SHARED((NACC, 128), _f32),
            pltpu.SemaphoreType.DMA,
            pltpu.SemaphoreType.DMA,
        ],
        name=name,
    )
    def scalar_pass(tab, zeros_hbm, rowidx, colidx, out,
                    row_v, col_v, buf0, buf1, acc, sem0, sem1):
        c = lax.axis_index("c")
        s = lax.axis_index("s")
        base = s * RPT
        pltpu.sync_copy(zeros_hbm, acc.at[pl.ds(base, RPT)])
        plsc.subcore_barrier()

        pltpu.sync_copy(rowidx.at[c, s], row_v)
        pltpu.sync_copy(colidx.at[c, s], col_v)
        pltpu.async_copy(tab.at[col_v.at[0]], buf0, sem0)

        def body(k, carry):
            j0 = 2 * k
            h1 = pltpu.async_copy(tab.at[col_v.at[j0 + 1]], buf1, sem1)
            pltpu.make_async_copy(tab.at[pl.ds(0, CH)], buf0, sem0).wait()
            pltpu.sync_copy(buf0, acc.at[row_v.at[j0]], add=True)

            @pl.when(k < IB // 2 - 1)
            def _():
                pltpu.async_copy(tab.at[col_v.at[j0 + 2]], buf0, sem0)

            h1.wait()
            pltpu.sync_copy(buf1, acc.at[row_v.at[j0 + 1]], add=True)
            return carry

        lax.fori_loop(0, IB // 2, body, 0)

        plsc.subcore_barrier()

        @pl.when(s < 15)
        def _():
            pltpu.sync_copy(acc.at[pl.ds(base, RPT)], out.at[c, pl.ds(base, RPT)])

        @pl.when(s == 15)
        def _():
            pltpu.sync_copy(acc.at[pl.ds(15 * RPT, N - 15 * RPT)],
                            out.at[c, pl.ds(15 * RPT, N - 15 * RPT)])

    return scalar_pass


_gcn_pass = _make_split_pass("gcn_edge_pass")
_scalar_pass = _make_split_pass("scalar_edge_pass")


def _k3_body(aggx_ref, gcw_ref, gctw_ref, gcb_ref, gctb_ref, a_ref,
             tabu_ref, tabt_ref, tabs_ref, repo_ref, g_ref, misc_ref):
    aggx = aggx_ref[0] + aggx_ref[1]
    rep_o = jnp.maximum(
        jnp.dot(aggx, gcw_ref[...], preferred_element_type=_f32)
        + gcb_ref[...], 0.0)
    rep_t = jnp.maximum(
        jnp.dot(aggx, gctw_ref[...], preferred_element_type=_f32)
        + gctb_ref[...], 0.0)
    a = a_ref[...]
    s1 = (jnp.dot(rep_o, a[0:128], preferred_element_type=_f32)
          + jnp.dot(rep_t, a[128:256], preferred_element_type=_f32))
    s2 = (jnp.dot(rep_o, a[256:384], preferred_element_type=_f32)
          + jnp.dot(rep_t, a[384:512], preferred_element_type=_f32))
    c = jnp.max(s2)
    e2 = jnp.exp(s2 - c)
    u = e2 * rep_t
    cid = lax.broadcasted_iota(jnp.int32, (N, 128), 1)
    sc = jnp.where(cid == 0, e2, jnp.where(cid == 1, 1.0, 0.0))
    zpad = jnp.zeros((NT - N, 128), _f32)
    tabu_ref[0:N, :] = u
    tabu_ref[N:NT, :] = zpad
    tabt_ref[0:N, :] = rep_t
    tabt_ref[N:NT, :] = zpad
    tabs_ref[0:N, :] = sc
    tabs_ref[N:NT, :] = zpad
    repo_ref[...] = rep_o
    g_ref[...] = s1 + c
    t_sum = jnp.sum(rep_t, axis=0, keepdims=True)
    misc_ref[...] = jnp.broadcast_to(t_sum, (8, 128))


def _k3(aggx, gc_W, gct_W, gc_b, gct_b, a):
    return pl.pallas_call(
        _k3_body,
        out_shape=[
            jax.ShapeDtypeStruct((NT, 128), _f32),
            jax.ShapeDtypeStruct((NT, 128), _f32),
            jax.ShapeDtypeStruct((NT, 128), _f32),
            jax.ShapeDtypeStruct((N, 128), _f32),
            jax.ShapeDtypeStruct((N, 1), _f32),
            jax.ShapeDtypeStruct((8, 128), _f32),
        ],
    )(aggx, gc_W, gct_W, gc_b.reshape(1, 128), gct_b.reshape(1, 128), a)


def _k5_body(att_ref, att2_ref, rept_ref, repo_ref, g_ref, misc_ref, t_ref,
             ppw_ref, ppb_ref, pp2w_ref, pp2b_ref,
             o00w_ref, o00b_ref, o10w_ref, o10b_ref,
             o01w_ref, o01b_ref, o11w_ref, o11b_ref,
             y_ref, out2_ref, treat_ref):
    v = att_ref[0]
    wm = att_ref[1]
    z = att2_ref[0, :, 0:1] + att2_ref[1, :, 0:1]
    deg = att2_ref[0, :, 1:2] + att2_ref[1, :, 1:2]
    g = g_ref[...]
    m = jnp.maximum(g, 0.0)
    amp = jnp.exp(0.0 - m)
    bmp = jnp.exp(g - m)
    t_sum = misc_ref[0:1, :]
    denom = (float(N) - deg) * amp + bmp * z
    out2 = (amp * (t_sum - wm) + bmp * v) / denom + repo_ref[...]
    out2_ref[...] = out2
    rep_t = rept_ref[...]
    tm = jnp.dot(rep_t, ppw_ref[...], preferred_element_type=_f32) + ppb_ref[...]
    logits = jnp.dot(tm, pp2w_ref[...], preferred_element_type=_f32) + pp2b_ref[...]
    treat_ref[...] = jax.nn.sigmoid(logits)
    y00 = jnp.maximum(jnp.dot(out2, o00w_ref[...], preferred_element_type=_f32)
                      + o00b_ref[...], 0.0)
    y0 = jnp.dot(y00, o01w_ref[...], preferred_element_type=_f32) + o01b_ref[...]
    y10 = jnp.maximum(jnp.dot(out2, o10w_ref[...], preferred_element_type=_f32)
                      + o10b_ref[...], 0.0)
    y1 = jnp.dot(y10, o11w_ref[...], preferred_element_type=_f32) + o11b_ref[...]
    y_ref[...] = jnp.where(t_ref[...] > 0, y1, y0)


def _k5(att, att2, rep_t, rep_o, g, misc, t2, pp_W, pp_b, pp2_W, pp2_b,
        o00_W, o00_b, o10_W, o10_b, o01_W, o01_b, o11_W, o11_b):
    nb = 10
    br = N // nb
    row_specs = lambda wdt: pl.BlockSpec((br, wdt), lambda i: (i, 0))
    pair_spec = pl.BlockSpec((2, br, 128), lambda i: (0, i, 0))
    full = lambda shape: pl.BlockSpec(shape, lambda i: tuple(0 for _ in shape))
    return pl.pallas_call(
        _k5_body,
        grid=(nb,),
        in_specs=[
            pair_spec, pair_spec,
            row_specs(128), row_specs(128), row_specs(1),
            full((8, 128)), row_specs(1),
            full((128, 128)), full((1, 128)), full((128, 2)), full((1, 2)),
            full((128, 128)), full((1, 128)), full((128, 128)), full((1, 128)),
            full((128, 1)), full((1, 1)), full((128, 1)), full((1, 1)),
        ],
        out_specs=[row_specs(1), row_specs(128), row_specs(2)],
        out_shape=[
            jax.ShapeDtypeStruct((N, 1), _f32),
            jax.ShapeDtypeStruct((N, 128), _f32),
            jax.ShapeDtypeStruct((N, 2), _f32),
        ],
    )(att, att2, rep_t, rep_o, g, misc, t2,
      pp_W, pp_b.reshape(1, 128), pp2_W, pp2_b.reshape(1, 2),
      o00_W, o00_b.reshape(1, 128), o10_W, o10_b.reshape(1, 128),
      o01_W, o01_b.reshape(1, 1), o11_W, o11_b.reshape(1, 1))


def kernel(x, edge_index, t, gc_W, gc_b, gct_W, gct_b, a, pp_W, pp_b, pp2_W, pp2_b, o00_W, o00_b, o10_W, o10_b, o01_W, o01_b, o11_W, o11_b):
    row = edge_index[0]
    col = edge_index[1]
    npad = EPAD - E

    pad_row = N + (jnp.arange(npad, dtype=jnp.int32) % (NACC - N))
    pad_col = jnp.arange(npad, dtype=jnp.int32) % (NACC - N)
    trash_e = N + (jnp.arange(E, dtype=jnp.int32) % (NACC - N))

    row_raw = jnp.concatenate([row, pad_row])
    col_raw = jnp.concatenate([col, pad_col])
    rowidx_gcn = row_raw.reshape(2, 16, IB, CH)
    col_gcn = col_raw.reshape(2, 16, IB, CH)

    key = (row * N + col
           + jnp.minimum(row_raw[0], 0)
           + jnp.minimum(col_raw[0], 0))
    ks = jnp.sort(key)
    rs = ks // N
    cs = ks - rs * N
    dup = jnp.concatenate([jnp.zeros((1,), bool), ks[1:] == ks[:-1]])
    rse = jnp.where(dup, trash_e, rs)

    rs_p = jnp.concatenate([rse, pad_row])
    cse_p = jnp.concatenate([cs, pad_col])
    rowidx = rs_p.reshape(16, TPC, CH)
    col_att = cse_p.reshape(16, TPC, CH)
    rowidx_h = rs_p.reshape(2, 16, IB, CH)
    col_att_h = cse_p.reshape(2, 16, IB, CH)

    zeros128 = jnp.zeros((RPT, 128), _f32)

    aggx = _gcn_pass(x, zeros128, rowidx_gcn, col_gcn)
    tab_u, tab_t, tab_s, rep_o, g, misc = _k3(aggx, gc_W, gct_W, gc_b, gct_b, a)
    att = _edge_pass_att(tab_u, tab_t, zeros128, rowidx, col_att)
    att2 = _scalar_pass(tab_s, zeros128, rowidx_h, col_att_h)

    rep_t = tab_t[0:N, :]
    y2, out2, treat = _k5(att, att2, rep_t, rep_o, g, misc, t.reshape(N, 1),
                          pp_W, pp_b, pp2_W, pp2_b,
                          o00_W, o00_b, o10_W, o10_b,
                          o01_W, o01_b, o11_W, o11_b)
    return (y2.reshape(-1), out2, treat)
